# baseline (speedup 1.0000x reference)
import functools

import jax
import jax.numpy as jnp
from jax import lax
from jax.experimental import pallas as pl
from jax.experimental.pallas import tpu as pltpu
from jax.experimental.pallas import tpu_sc as plsc

_N = 10000
_E = 320000
_D = 128
_H = 64
_B = 8192

_W1 = 80
_NC = 2
_NS = 16
_NW = _NC * _NS
_EPW = _E // _NW
_CH = 80
_NCHK = _EPW // _CH
_NP = 10240
_RPT = _NP // _NS
_BPW = _B // _NW
_NBUF = 5

_mesh = plsc.VectorSubcoreMesh(core_axis_name="c", subcore_axis_name="s")


def _make_segsum(width):
    @functools.partial(
        pl.kernel,
        out_type=jax.ShapeDtypeStruct((_NC, _NP, width), jnp.float32),
        mesh=_mesh,
        compiler_params=pltpu.CompilerParams(use_tc_tiling_on_sc=False),
        scratch_types=[
            pltpu.VMEM((_NCHK, _CH), jnp.int32),
            pltpu.VMEM((_NCHK, _CH), jnp.int32),
            pltpu.VMEM((_NBUF, _CH, width), jnp.float32),
            pltpu.VMEM_SHARED((_NP, width), jnp.float32),
            pltpu.SemaphoreType.DMA((_NBUF,)),
            pltpu.SemaphoreType.DMA((_NBUF,)),
        ],
    )
    def seg(s_hbm, d_hbm, tab_hbm, zeros_hbm, out_hbm,
            sidx_v, didx_v, rows_v, acc_sh, gsem, ssem):
        cid = lax.axis_index("c")
        sid = lax.axis_index("s")
        wid = cid * _NS + sid
        pltpu.sync_copy(zeros_hbm.at[pl.ds(sid * _RPT, _RPT)],
                        acc_sh.at[pl.ds(sid * _RPT, _RPT)])
        pltpu.sync_copy(s_hbm.at[wid], sidx_v)
        pltpu.sync_copy(d_hbm.at[wid], didx_v)
        plsc.subcore_barrier()

        for b in range(_NBUF):
            pltpu.async_copy(tab_hbm.at[sidx_v.at[b]], rows_v.at[b], gsem.at[b])

        def outer(t, carry):
            j0 = t * _NBUF
            for b in range(_NBUF):
                pltpu.make_async_copy(
                    tab_hbm.at[sidx_v.at[j0 + b]], rows_v.at[b], gsem.at[b]).wait()
                pltpu.async_copy(
                    rows_v.at[b], acc_sh.at[didx_v.at[j0 + b]], ssem.at[b], add=True)
            for b in range(_NBUF):
                nj = j0 + _NBUF + b

                @pl.when(nj < _NCHK)
                def _():
                    pltpu.make_async_copy(
                        rows_v.at[b], acc_sh.at[didx_v.at[j0 + b]], ssem.at[b]).wait()
                    pltpu.async_copy(
                        tab_hbm.at[sidx_v.at[nj]], rows_v.at[b], gsem.at[b])
            return carry

        lax.fori_loop(0, _NCHK // _NBUF, outer, 0)
        jlast = _NCHK - _NBUF
        for b in range(_NBUF):
            pltpu.make_async_copy(
                rows_v.at[b], acc_sh.at[didx_v.at[jlast + b]], ssem.at[b]).wait()
        plsc.subcore_barrier()
        pltpu.sync_copy(acc_sh.at[pl.ds(sid * _RPT, _RPT)],
                        out_hbm.at[cid, pl.ds(sid * _RPT, _RPT)])

    return seg


_segsum80 = _make_segsum(_W1)
_segsum64 = _make_segsum(_H)

_SLAB = 128
_NSLAB = _RPT // _SLAB


@functools.partial(
    pl.kernel,
    out_type=jax.ShapeDtypeStruct((_B,), jnp.float32),
    mesh=_mesh,
    compiler_params=pltpu.CompilerParams(
        use_tc_tiling_on_sc=False, needs_layout_passes=False),
    scratch_types=[
        pltpu.VMEM((2, 128), jnp.int32),
        pltpu.VMEM((2, 128), jnp.int32),
        pltpu.VMEM((_SLAB, _H), jnp.float32),
        pltpu.VMEM((_SLAB, _H), jnp.float32),
        pltpu.VMEM((_SLAB, _W1), jnp.float32),
        pltpu.VMEM((_SLAB, _H), jnp.float32),
        pltpu.VMEM_SHARED((_NP, _H), jnp.float32),
        pltpu.VMEM((_BPW, _H), jnp.float32),
        pltpu.VMEM((_BPW, _H), jnp.float32),
        pltpu.VMEM((_BPW, 16), jnp.float32),
        pltpu.VMEM((_BPW,), jnp.float32),
        pltpu.SemaphoreType.DMA((4,)),
    ],
)
def _decode(si_hbm, di_hbm, aggp_hbm, r2e_hbm, out_hbm,
            si_v, di_v, pa_v, pb_v, re_v, z_v, zsh, zs_v, zd_v, stage_v, o_v,
            sems):
    cid = lax.axis_index("c")
    sid = lax.axis_index("s")
    wid = cid * _NS + sid
    pltpu.sync_copy(si_hbm.at[wid], si_v)
    pltpu.sync_copy(di_hbm.at[wid], di_v)

    for k in range(_NSLAB):
        r0 = sid * _RPT + k * _SLAB
        pltpu.sync_copy(aggp_hbm.at[0, pl.ds(r0, _SLAB)], pa_v)
        pltpu.sync_copy(aggp_hbm.at[1, pl.ds(r0, _SLAB)], pb_v)
        pltpu.sync_copy(r2e_hbm.at[pl.ds(r0, _SLAB)], re_v)

        def zrow(p, carry):
            iv = re_v[p, pl.ds(_H, 16)]
            ivec = jnp.zeros((16,), jnp.float32) + iv[0]
            for c in range(_H // 16):
                sl = pl.ds(c * 16, 16)
                zc = (pa_v[p, sl] + pb_v[p, sl]) * ivec + re_v[p, sl]
                z_v[p, sl] = jnp.maximum(zc, 0.0)
            return carry

        lax.fori_loop(0, _SLAB, zrow, 0)
        pltpu.sync_copy(z_v, zsh.at[pl.ds(r0, _SLAB)])

    plsc.subcore_barrier()

    for t in range(2):
        pltpu.async_copy(zsh.at[si_v.at[t]], zs_v.at[pl.ds(t * 128, 128)],
                         sems.at[t])
        pltpu.async_copy(zsh.at[di_v.at[t]], zd_v.at[pl.ds(t * 128, 128)],
                         sems.at[2 + t])
    for t in range(2):
        pltpu.make_async_copy(zsh.at[si_v.at[t]], zs_v.at[pl.ds(t * 128, 128)],
                              sems.at[t]).wait()
        pltpu.make_async_copy(zsh.at[di_v.at[t]], zd_v.at[pl.ds(t * 128, 128)],
                              sems.at[2 + t]).wait()

    def body(p, carry):
        t = ((zs_v[p, pl.ds(0, 16)] * zd_v[p, pl.ds(0, 16)]
              + zs_v[p, pl.ds(16, 16)] * zd_v[p, pl.ds(16, 16)])
             + (zs_v[p, pl.ds(32, 16)] * zd_v[p, pl.ds(32, 16)]
                + zs_v[p, pl.ds(48, 16)] * zd_v[p, pl.ds(48, 16)]))
        stage_v[p, pl.ds(0, 16)] = jnp.cumsum(t)
        return carry

    lax.fori_loop(0, _BPW, body, 0)

    lanes = lax.iota(jnp.int32, 16)
    c15 = jnp.zeros((16,), jnp.int32) + 15

    def sig(g, carry):
        v = plsc.load_gather(stage_v, [g * 16 + lanes, c15])
        o_v[pl.ds(g * 16, 16)] = 1.0 / (1.0 + jnp.exp(-v))
        return carry

    lax.fori_loop(0, _BPW // 16, sig, 0)
    pltpu.sync_copy(o_v, out_hbm.at[pl.ds(wid * _BPW, _BPW)])


_CT = (((1,), (1,)), ((), ()))


def _tc_dense1(x, W1le, e1, W1r, b1):
    g = 10
    bn = _N // g

    def body(x_ref, wle_ref, e1_ref, wr_ref, b1_ref, yext_ref, r1_ref):
        xb = x_ref[...]
        yext_ref[...] = lax.dot_general(
            xb, wle_ref[...], _CT, preferred_element_type=jnp.float32) + e1_ref[...]
        r1_ref[...] = lax.dot_general(
            xb, wr_ref[...], _CT, preferred_element_type=jnp.float32) + b1_ref[...]

    return pl.pallas_call(
        body,
        grid=(g,),
        in_specs=[pl.BlockSpec((bn, _D), lambda i: (i, 0)),
                  pl.BlockSpec((_W1, _D), lambda i: (0, 0)),
                  pl.BlockSpec((1, _W1), lambda i: (0, 0)),
                  pl.BlockSpec((_H, _D), lambda i: (0, 0)),
                  pl.BlockSpec((1, _H), lambda i: (0, 0))],
        out_specs=[pl.BlockSpec((bn, _W1), lambda i: (i, 0)),
                   pl.BlockSpec((bn, _H), lambda i: (i, 0))],
        out_shape=[jax.ShapeDtypeStruct((_N, _W1), jnp.float32),
                   jax.ShapeDtypeStruct((_N, _H), jnp.float32)],
    )(x, W1le, e1, W1r, b1)


def _tc_dense2(agg1p, r1, W2l, b2, W2r):
    g = 10
    bn = _N // g

    def body(aggp_ref, r1_ref, wl_ref, b_ref, wr_ref, y2_ref, r2e_ref):
        a = aggp_ref[...]
        agg = a[0] + a[1]
        inv = 1.0 / jnp.maximum(agg[:, _H:_H + 1], 1.0)
        h = jnp.maximum(agg[:, :_H] * inv + r1_ref[...], 0.0)
        y2_ref[...] = lax.dot_general(
            h, wl_ref[...], _CT, preferred_element_type=jnp.float32)
        r2 = lax.dot_general(
            h, wr_ref[...], _CT, preferred_element_type=jnp.float32) + b_ref[...]
        lane = lax.broadcasted_iota(jnp.int32, (bn, _W1 - _H), 1)
        ext = jnp.where(lane == 0, inv, 0.0)
        r2e_ref[...] = jnp.concatenate([r2, ext], axis=1)

    return pl.pallas_call(
        body,
        grid=(g,),
        in_specs=[pl.BlockSpec((_NC, bn, _W1), lambda i: (0, i, 0)),
                  pl.BlockSpec((bn, _H), lambda i: (i, 0)),
                  pl.BlockSpec((_H, _H), lambda i: (0, 0)),
                  pl.BlockSpec((1, _H), lambda i: (0, 0)),
                  pl.BlockSpec((_H, _H), lambda i: (0, 0))],
        out_specs=[pl.BlockSpec((bn, _H), lambda i: (i, 0)),
                   pl.BlockSpec((bn, _W1), lambda i: (i, 0))],
        out_shape=[jax.ShapeDtypeStruct((_N, _H), jnp.float32),
                   jax.ShapeDtypeStruct((_NP, _W1), jnp.float32)],
    )(agg1p, r1, W2l, b2, W2r)


def kernel(x, edge_index, src, dst, W1l, b1l, W1r, W2l, b2l, W2r):
    f32 = jnp.float32
    W1le = jnp.zeros((_W1, _D), f32).at[:_H].set(W1l)
    e1 = jnp.zeros((1, _W1), f32).at[0, _H].set(1.0)

    s_r = edge_index[0].reshape(_NW, _NCHK, _CH)
    d_r = edge_index[1].reshape(_NW, _NCHK, _CH)
    zeros1 = jnp.zeros((_NP, _W1), f32)
    zeros2 = jnp.zeros((_NP, _H), f32)

    yext, r1 = _tc_dense1(x, W1le, e1, W1r, b1l.reshape(1, _H))
    agg1p = _segsum80(s_r, d_r, yext, zeros1)
    y2, r2e = _tc_dense2(agg1p, r1, W2l, b2l.reshape(1, _H), W2r)
    agg2p = _segsum64(s_r, d_r, y2, zeros2)
    return _decode(src.reshape(_NW, 2, 128), dst.reshape(_NW, 2, 128),
                   agg2p, r2e)


# speedup vs baseline: 1.0484x; 1.0484x over previous
import functools

import jax
import jax.numpy as jnp
from jax import lax
from jax.experimental import pallas as pl
from jax.experimental.pallas import tpu as pltpu
from jax.experimental.pallas import tpu_sc as plsc

_N = 10000
_E = 320000
_D = 128
_H = 64
_B = 8192

_W1 = 80
_NC = 2
_NS = 16
_NW = _NC * _NS
_EPW = _E // _NW
_CH = 80
_NCHK = _EPW // _CH
_NP = 10240
_RPT = _NP // _NS
_BPW = _B // _NW
_NBUF = 5

_mesh = plsc.VectorSubcoreMesh(core_axis_name="c", subcore_axis_name="s")


def _make_segsum(width):
    @functools.partial(
        pl.kernel,
        out_type=jax.ShapeDtypeStruct((_NC, _NP, width), jnp.float32),
        mesh=_mesh,
        compiler_params=pltpu.CompilerParams(use_tc_tiling_on_sc=False),
        scratch_types=[
            pltpu.VMEM((_NCHK, _CH), jnp.int32),
            pltpu.VMEM((_NCHK, _CH), jnp.int32),
            pltpu.VMEM((_NBUF, _CH, width), jnp.float32),
            pltpu.VMEM_SHARED((_NP, width), jnp.float32),
            pltpu.SemaphoreType.DMA((_NBUF,)),
            pltpu.SemaphoreType.DMA((_NBUF,)),
        ],
    )
    def seg(s_hbm, d_hbm, tab_hbm, zeros_hbm, out_hbm,
            sidx_v, didx_v, rows_v, acc_sh, gsem, ssem):
        cid = lax.axis_index("c")
        sid = lax.axis_index("s")
        wid = cid * _NS + sid
        pltpu.sync_copy(zeros_hbm.at[pl.ds(sid * _RPT, _RPT)],
                        acc_sh.at[pl.ds(sid * _RPT, _RPT)])
        pltpu.sync_copy(s_hbm.at[wid], sidx_v)
        pltpu.sync_copy(d_hbm.at[wid], didx_v)
        plsc.subcore_barrier()

        for b in range(_NBUF):
            pltpu.async_copy(tab_hbm.at[sidx_v.at[b]], rows_v.at[b], gsem.at[b])

        def outer(t, carry):
            j0 = t * _NBUF
            for b in range(_NBUF):
                pltpu.make_async_copy(
                    tab_hbm.at[sidx_v.at[j0 + b]], rows_v.at[b], gsem.at[b]).wait()
                pltpu.async_copy(
                    rows_v.at[b], acc_sh.at[didx_v.at[j0 + b]], ssem.at[b], add=True)
            for b in range(_NBUF):
                nj = j0 + _NBUF + b

                @pl.when(nj < _NCHK)
                def _():
                    pltpu.make_async_copy(
                        rows_v.at[b], acc_sh.at[didx_v.at[j0 + b]], ssem.at[b]).wait()
                    pltpu.async_copy(
                        tab_hbm.at[sidx_v.at[nj]], rows_v.at[b], gsem.at[b])
            return carry

        lax.fori_loop(0, _NCHK // _NBUF, outer, 0)
        jlast = _NCHK - _NBUF
        for b in range(_NBUF):
            pltpu.make_async_copy(
                rows_v.at[b], acc_sh.at[didx_v.at[jlast + b]], ssem.at[b]).wait()
        plsc.subcore_barrier()
        pltpu.sync_copy(acc_sh.at[pl.ds(sid * _RPT, _RPT)],
                        out_hbm.at[cid, pl.ds(sid * _RPT, _RPT)])

    return seg


_segsum80 = _make_segsum(_W1)
_segsum64 = _make_segsum(_H)

_SLAB = 64
_NSLAB = _RPT // _SLAB


@functools.partial(
    pl.kernel,
    out_type=jax.ShapeDtypeStruct((_B,), jnp.float32),
    mesh=_mesh,
    compiler_params=pltpu.CompilerParams(
        use_tc_tiling_on_sc=False, needs_layout_passes=False),
    scratch_types=[
        pltpu.VMEM((2, 128), jnp.int32),
        pltpu.VMEM((2, 128), jnp.int32),
        pltpu.VMEM((2, _SLAB, _H), jnp.float32),
        pltpu.VMEM((2, _SLAB, _H), jnp.float32),
        pltpu.VMEM((2, _SLAB, _W1), jnp.float32),
        pltpu.VMEM((_SLAB, _H), jnp.float32),
        pltpu.VMEM_SHARED((_NP, _H), jnp.float32),
        pltpu.VMEM((_BPW, _H), jnp.float32),
        pltpu.VMEM((_BPW, _H), jnp.float32),
        pltpu.VMEM((_BPW, 16), jnp.float32),
        pltpu.VMEM((_BPW,), jnp.float32),
        pltpu.SemaphoreType.DMA((4,)),
        pltpu.SemaphoreType.DMA((2, 3)),
    ],
)
def _decode(si_hbm, di_hbm, aggp_hbm, r2e_hbm, out_hbm,
            si_v, di_v, pa_v, pb_v, re_v, z_v, zsh, zs_v, zd_v, stage_v, o_v,
            sems, lsem):
    cid = lax.axis_index("c")
    sid = lax.axis_index("s")
    wid = cid * _NS + sid
    pltpu.sync_copy(si_hbm.at[wid], si_v)
    pltpu.sync_copy(di_hbm.at[wid], di_v)

    def _issue(k, buf):
        r0 = sid * _RPT + k * _SLAB
        pltpu.async_copy(aggp_hbm.at[0, pl.ds(r0, _SLAB)], pa_v.at[buf],
                         lsem.at[buf, 0])
        pltpu.async_copy(aggp_hbm.at[1, pl.ds(r0, _SLAB)], pb_v.at[buf],
                         lsem.at[buf, 1])
        pltpu.async_copy(r2e_hbm.at[pl.ds(r0, _SLAB)], re_v.at[buf],
                         lsem.at[buf, 2])

    def _wait(k, buf):
        r0 = sid * _RPT + k * _SLAB
        pltpu.make_async_copy(aggp_hbm.at[0, pl.ds(r0, _SLAB)], pa_v.at[buf],
                              lsem.at[buf, 0]).wait()
        pltpu.make_async_copy(aggp_hbm.at[1, pl.ds(r0, _SLAB)], pb_v.at[buf],
                              lsem.at[buf, 1]).wait()
        pltpu.make_async_copy(r2e_hbm.at[pl.ds(r0, _SLAB)], re_v.at[buf],
                              lsem.at[buf, 2]).wait()

    _issue(0, 0)
    for k in range(_NSLAB):
        buf = k % 2
        if k + 1 < _NSLAB:
            _issue(k + 1, 1 - buf)
        _wait(k, buf)

        def zrow(p, carry):
            iv = re_v[buf, p, pl.ds(_H, 16)]
            ivec = jnp.zeros((16,), jnp.float32) + iv[0]
            for c in range(_H // 16):
                sl = pl.ds(c * 16, 16)
                zc = (pa_v[buf, p, sl] + pb_v[buf, p, sl]) * ivec + re_v[buf, p, sl]
                z_v[p, sl] = jnp.maximum(zc, 0.0)
            return carry

        lax.fori_loop(0, _SLAB, zrow, 0)
        pltpu.sync_copy(z_v, zsh.at[pl.ds(sid * _RPT + k * _SLAB, _SLAB)])

    plsc.subcore_barrier()

    for t in range(2):
        pltpu.async_copy(zsh.at[si_v.at[t]], zs_v.at[pl.ds(t * 128, 128)],
                         sems.at[t])
        pltpu.async_copy(zsh.at[di_v.at[t]], zd_v.at[pl.ds(t * 128, 128)],
                         sems.at[2 + t])
    for t in range(2):
        pltpu.make_async_copy(zsh.at[si_v.at[t]], zs_v.at[pl.ds(t * 128, 128)],
                              sems.at[t]).wait()
        pltpu.make_async_copy(zsh.at[di_v.at[t]], zd_v.at[pl.ds(t * 128, 128)],
                              sems.at[2 + t]).wait()

    def body(p, carry):
        t = ((zs_v[p, pl.ds(0, 16)] * zd_v[p, pl.ds(0, 16)]
              + zs_v[p, pl.ds(16, 16)] * zd_v[p, pl.ds(16, 16)])
             + (zs_v[p, pl.ds(32, 16)] * zd_v[p, pl.ds(32, 16)]
                + zs_v[p, pl.ds(48, 16)] * zd_v[p, pl.ds(48, 16)]))
        stage_v[p, pl.ds(0, 16)] = jnp.cumsum(t)
        return carry

    lax.fori_loop(0, _BPW, body, 0)

    lanes = lax.iota(jnp.int32, 16)
    c15 = jnp.zeros((16,), jnp.int32) + 15

    def sig(g, carry):
        v = plsc.load_gather(stage_v, [g * 16 + lanes, c15])
        o_v[pl.ds(g * 16, 16)] = 1.0 / (1.0 + jnp.exp(-v))
        return carry

    lax.fori_loop(0, _BPW // 16, sig, 0)
    pltpu.sync_copy(o_v, out_hbm.at[pl.ds(wid * _BPW, _BPW)])


_CT = (((1,), (1,)), ((), ()))


def _tc_dense1(x, W1le, e1, W1r, b1):
    g = 10
    bn = _N // g

    def body(x_ref, wle_ref, e1_ref, wr_ref, b1_ref, yext_ref, r1_ref):
        xb = x_ref[...]
        yext_ref[...] = lax.dot_general(
            xb, wle_ref[...], _CT, preferred_element_type=jnp.float32) + e1_ref[...]
        r1_ref[...] = lax.dot_general(
            xb, wr_ref[...], _CT, preferred_element_type=jnp.float32) + b1_ref[...]

    return pl.pallas_call(
        body,
        grid=(g,),
        in_specs=[pl.BlockSpec((bn, _D), lambda i: (i, 0)),
                  pl.BlockSpec((_W1, _D), lambda i: (0, 0)),
                  pl.BlockSpec((1, _W1), lambda i: (0, 0)),
                  pl.BlockSpec((_H, _D), lambda i: (0, 0)),
                  pl.BlockSpec((1, _H), lambda i: (0, 0))],
        out_specs=[pl.BlockSpec((bn, _W1), lambda i: (i, 0)),
                   pl.BlockSpec((bn, _H), lambda i: (i, 0))],
        out_shape=[jax.ShapeDtypeStruct((_N, _W1), jnp.float32),
                   jax.ShapeDtypeStruct((_N, _H), jnp.float32)],
    )(x, W1le, e1, W1r, b1)


def _tc_dense2(agg1p, r1, W2l, b2, W2r):
    g = 10
    bn = _N // g

    def body(aggp_ref, r1_ref, wl_ref, b_ref, wr_ref, y2_ref, r2e_ref):
        a = aggp_ref[...]
        agg = a[0] + a[1]
        inv = 1.0 / jnp.maximum(agg[:, _H:_H + 1], 1.0)
        h = jnp.maximum(agg[:, :_H] * inv + r1_ref[...], 0.0)
        y2_ref[...] = lax.dot_general(
            h, wl_ref[...], _CT, preferred_element_type=jnp.float32)
        r2 = lax.dot_general(
            h, wr_ref[...], _CT, preferred_element_type=jnp.float32) + b_ref[...]
        lane = lax.broadcasted_iota(jnp.int32, (bn, _W1 - _H), 1)
        ext = jnp.where(lane == 0, inv, 0.0)
        r2e_ref[...] = jnp.concatenate([r2, ext], axis=1)

    return pl.pallas_call(
        body,
        grid=(g,),
        in_specs=[pl.BlockSpec((_NC, bn, _W1), lambda i: (0, i, 0)),
                  pl.BlockSpec((bn, _H), lambda i: (i, 0)),
                  pl.BlockSpec((_H, _H), lambda i: (0, 0)),
                  pl.BlockSpec((1, _H), lambda i: (0, 0)),
                  pl.BlockSpec((_H, _H), lambda i: (0, 0))],
        out_specs=[pl.BlockSpec((bn, _H), lambda i: (i, 0)),
                   pl.BlockSpec((bn, _W1), lambda i: (i, 0))],
        out_shape=[jax.ShapeDtypeStruct((_N, _H), jnp.float32),
                   jax.ShapeDtypeStruct((_NP, _W1), jnp.float32)],
    )(agg1p, r1, W2l, b2, W2r)


def kernel(x, edge_index, src, dst, W1l, b1l, W1r, W2l, b2l, W2r):
    f32 = jnp.float32
    W1le = jnp.zeros((_W1, _D), f32).at[:_H].set(W1l)
    e1 = jnp.zeros((1, _W1), f32).at[0, _H].set(1.0)

    s_r = edge_index[0].reshape(_NW, _NCHK, _CH)
    d_r = edge_index[1].reshape(_NW, _NCHK, _CH)
    zeros1 = jnp.zeros((_NP, _W1), f32)
    zeros2 = jnp.zeros((_NP, _H), f32)

    yext, r1 = _tc_dense1(x, W1le, e1, W1r, b1l.reshape(1, _H))
    agg1p = _segsum80(s_r, d_r, yext, zeros1)
    y2, r2e = _tc_dense2(agg1p, r1, W2l, b2l.reshape(1, _H), W2r)
    agg2p = _segsum64(s_r, d_r, y2, zeros2)
    return _decode(src.reshape(_NW, 2, 128), dst.reshape(_NW, 2, 128),
                   agg2p, r2e)


# speedup vs baseline: 1.0759x; 1.0263x over previous
import functools

import jax
import jax.numpy as jnp
from jax import lax
from jax.experimental import pallas as pl
from jax.experimental.pallas import tpu as pltpu
from jax.experimental.pallas import tpu_sc as plsc

_N = 10000
_E = 320000
_D = 128
_H = 64
_B = 8192

_W1 = 80
_NC = 2
_NS = 16
_NW = _NC * _NS
_EPW = _E // _NW
_CH = 80
_NCHK = _EPW // _CH
_NP = 10240
_RPT = _NP // _NS
_BPW = _B // _NW
_NBUF = 5

_mesh = plsc.VectorSubcoreMesh(core_axis_name="c", subcore_axis_name="s")


def _make_segsum(width):
    @functools.partial(
        pl.kernel,
        out_type=(jax.ShapeDtypeStruct((_NP, width), jnp.float32),
                  jax.ShapeDtypeStruct((_NP, width), jnp.float32)),
        mesh=_mesh,
        compiler_params=pltpu.CompilerParams(use_tc_tiling_on_sc=False),
        scratch_types=[
            pltpu.VMEM((_NCHK, _CH), jnp.int32),
            pltpu.VMEM((_NCHK, _CH), jnp.int32),
            pltpu.VMEM((_NBUF, _CH, width), jnp.float32),
            pltpu.VMEM_SHARED((_NP, width), jnp.float32),
            pltpu.SemaphoreType.DMA((_NBUF,)),
            pltpu.SemaphoreType.DMA((_NBUF,)),
        ],
    )
    def seg(s_hbm, d_hbm, tab_hbm, zeros_hbm, out0_hbm, out1_hbm,
            sidx_v, didx_v, rows_v, acc_sh, gsem, ssem):
        cid = lax.axis_index("c")
        sid = lax.axis_index("s")
        wid = cid * _NS + sid
        pltpu.sync_copy(zeros_hbm.at[pl.ds(sid * _RPT, _RPT)],
                        acc_sh.at[pl.ds(sid * _RPT, _RPT)])
        pltpu.sync_copy(s_hbm.at[wid], sidx_v)
        pltpu.sync_copy(d_hbm.at[wid], didx_v)
        plsc.subcore_barrier()

        for b in range(_NBUF):
            pltpu.async_copy(tab_hbm.at[sidx_v.at[b]], rows_v.at[b], gsem.at[b])

        def outer(t, carry):
            j0 = t * _NBUF
            for b in range(_NBUF):
                pltpu.make_async_copy(
                    tab_hbm.at[sidx_v.at[j0 + b]], rows_v.at[b], gsem.at[b]).wait()
                pltpu.async_copy(
                    rows_v.at[b], acc_sh.at[didx_v.at[j0 + b]], ssem.at[b], add=True)
            for b in range(_NBUF):
                nj = j0 + _NBUF + b

                @pl.when(nj < _NCHK)
                def _():
                    pltpu.make_async_copy(
                        rows_v.at[b], acc_sh.at[didx_v.at[j0 + b]], ssem.at[b]).wait()
                    pltpu.async_copy(
                        tab_hbm.at[sidx_v.at[nj]], rows_v.at[b], gsem.at[b])
            return carry

        lax.fori_loop(0, _NCHK // _NBUF, outer, 0)
        jlast = _NCHK - _NBUF
        for b in range(_NBUF):
            pltpu.make_async_copy(
                rows_v.at[b], acc_sh.at[didx_v.at[jlast + b]], ssem.at[b]).wait()
        plsc.subcore_barrier()

        @pl.when(cid == 0)
        def _():
            pltpu.sync_copy(acc_sh.at[pl.ds(sid * _RPT, _RPT)],
                            out0_hbm.at[pl.ds(sid * _RPT, _RPT)])

        @pl.when(cid == 1)
        def _():
            pltpu.sync_copy(acc_sh.at[pl.ds(sid * _RPT, _RPT)],
                            out1_hbm.at[pl.ds(sid * _RPT, _RPT)])

    return seg


_segsum80 = _make_segsum(_W1)
_segsum64 = _make_segsum(_H)

@functools.partial(
    pl.kernel,
    out_type=jax.ShapeDtypeStruct((_B,), jnp.float32),
    mesh=_mesh,
    compiler_params=pltpu.CompilerParams(
        use_tc_tiling_on_sc=False, needs_layout_passes=False),
    scratch_types=[
        pltpu.VMEM((2, 128), jnp.int32),
        pltpu.VMEM((2, 128), jnp.int32),
        pltpu.VMEM((_BPW, _H), jnp.float32),
        pltpu.VMEM((_BPW, _H), jnp.float32),
        pltpu.VMEM((_BPW, _W1), jnp.float32),
        pltpu.VMEM((_BPW, _H), jnp.float32),
        pltpu.VMEM((_BPW, _H), jnp.float32),
        pltpu.VMEM((_BPW, _W1), jnp.float32),
        pltpu.VMEM((_BPW, 16), jnp.float32),
        pltpu.VMEM((_BPW,), jnp.float32),
        pltpu.SemaphoreType.DMA((12,)),
    ],
)
def _decode(si_hbm, di_hbm, p0_hbm, p1_hbm, r2e_hbm, out_hbm,
            si_v, di_v, as0_v, as1_v, rs_v, ad0_v, ad1_v, rd_v, stage_v, o_v,
            sems):
    cid = lax.axis_index("c")
    sid = lax.axis_index("s")
    wid = cid * _NS + sid
    pltpu.sync_copy(si_hbm.at[wid], si_v)
    pltpu.sync_copy(di_hbm.at[wid], di_v)

    def _copies():
        cps = []
        for t in range(2):
            sl = pl.ds(t * 128, 128)
            cps += [(p0_hbm.at[si_v.at[t]], as0_v.at[sl]),
                    (p1_hbm.at[si_v.at[t]], as1_v.at[sl]),
                    (r2e_hbm.at[si_v.at[t]], rs_v.at[sl]),
                    (p0_hbm.at[di_v.at[t]], ad0_v.at[sl]),
                    (p1_hbm.at[di_v.at[t]], ad1_v.at[sl]),
                    (r2e_hbm.at[di_v.at[t]], rd_v.at[sl])]
        return cps

    for i, (s, d) in enumerate(_copies()):
        pltpu.async_copy(s, d, sems.at[i])
    for i, (s, d) in enumerate(_copies()):
        pltpu.make_async_copy(s, d, sems.at[i]).wait()

    def zrow(p, carry):
        ivs = rs_v[p, pl.ds(_H, 16)]
        ivd = rd_v[p, pl.ds(_H, 16)]
        vs = jnp.zeros((16,), jnp.float32) + ivs[0]
        vd = jnp.zeros((16,), jnp.float32) + ivd[0]
        for c in range(_H // 16):
            sl = pl.ds(c * 16, 16)
            zs = (as0_v[p, sl] + as1_v[p, sl]) * vs + rs_v[p, sl]
            as0_v[p, sl] = jnp.maximum(zs, 0.0)
            zd = (ad0_v[p, sl] + ad1_v[p, sl]) * vd + rd_v[p, sl]
            ad0_v[p, sl] = jnp.maximum(zd, 0.0)
        return carry

    lax.fori_loop(0, _BPW, zrow, 0)

    def body(p, carry):
        t = ((as0_v[p, pl.ds(0, 16)] * ad0_v[p, pl.ds(0, 16)]
              + as0_v[p, pl.ds(16, 16)] * ad0_v[p, pl.ds(16, 16)])
             + (as0_v[p, pl.ds(32, 16)] * ad0_v[p, pl.ds(32, 16)]
                + as0_v[p, pl.ds(48, 16)] * ad0_v[p, pl.ds(48, 16)]))
        stage_v[p, pl.ds(0, 16)] = jnp.cumsum(t)
        return carry

    lax.fori_loop(0, _BPW, body, 0)

    lanes = lax.iota(jnp.int32, 16)
    c15 = jnp.zeros((16,), jnp.int32) + 15

    def sig(g, carry):
        v = plsc.load_gather(stage_v, [g * 16 + lanes, c15])
        o_v[pl.ds(g * 16, 16)] = 1.0 / (1.0 + jnp.exp(-v))
        return carry

    lax.fori_loop(0, _BPW // 16, sig, 0)
    pltpu.sync_copy(o_v, out_hbm.at[pl.ds(wid * _BPW, _BPW)])


_CT = (((1,), (1,)), ((), ()))


def _tc_dense1(x, W1le, e1, W1r, b1):
    g = 10
    bn = _N // g

    def body(x_ref, wle_ref, e1_ref, wr_ref, b1_ref, yext_ref, r1_ref):
        xb = x_ref[...]
        yext_ref[...] = lax.dot_general(
            xb, wle_ref[...], _CT, preferred_element_type=jnp.float32) + e1_ref[...]
        r1_ref[...] = lax.dot_general(
            xb, wr_ref[...], _CT, preferred_element_type=jnp.float32) + b1_ref[...]

    return pl.pallas_call(
        body,
        grid=(g,),
        in_specs=[pl.BlockSpec((bn, _D), lambda i: (i, 0)),
                  pl.BlockSpec((_W1, _D), lambda i: (0, 0)),
                  pl.BlockSpec((1, _W1), lambda i: (0, 0)),
                  pl.BlockSpec((_H, _D), lambda i: (0, 0)),
                  pl.BlockSpec((1, _H), lambda i: (0, 0))],
        out_specs=[pl.BlockSpec((bn, _W1), lambda i: (i, 0)),
                   pl.BlockSpec((bn, _H), lambda i: (i, 0))],
        out_shape=[jax.ShapeDtypeStruct((_N, _W1), jnp.float32),
                   jax.ShapeDtypeStruct((_N, _H), jnp.float32)],
    )(x, W1le, e1, W1r, b1)


def _tc_dense2(a0, a1, r1, W2l, b2, W2r):
    g = 10
    bn = _N // g

    def body(a0_ref, a1_ref, r1_ref, wl_ref, b_ref, wr_ref, y2_ref, r2e_ref):
        agg = a0_ref[...] + a1_ref[...]
        inv = 1.0 / jnp.maximum(agg[:, _H:_H + 1], 1.0)
        h = jnp.maximum(agg[:, :_H] * inv + r1_ref[...], 0.0)
        y2_ref[...] = lax.dot_general(
            h, wl_ref[...], _CT, preferred_element_type=jnp.float32)
        r2 = lax.dot_general(
            h, wr_ref[...], _CT, preferred_element_type=jnp.float32) + b_ref[...]
        lane = lax.broadcasted_iota(jnp.int32, (bn, _W1 - _H), 1)
        ext = jnp.where(lane == 0, inv, 0.0)
        r2e_ref[...] = jnp.concatenate([r2, ext], axis=1)

    return pl.pallas_call(
        body,
        grid=(g,),
        in_specs=[pl.BlockSpec((bn, _W1), lambda i: (i, 0)),
                  pl.BlockSpec((bn, _W1), lambda i: (i, 0)),
                  pl.BlockSpec((bn, _H), lambda i: (i, 0)),
                  pl.BlockSpec((_H, _H), lambda i: (0, 0)),
                  pl.BlockSpec((1, _H), lambda i: (0, 0)),
                  pl.BlockSpec((_H, _H), lambda i: (0, 0))],
        out_specs=[pl.BlockSpec((bn, _H), lambda i: (i, 0)),
                   pl.BlockSpec((bn, _W1), lambda i: (i, 0))],
        out_shape=[jax.ShapeDtypeStruct((_N, _H), jnp.float32),
                   jax.ShapeDtypeStruct((_NP, _W1), jnp.float32)],
    )(a0, a1, r1, W2l, b2, W2r)


def kernel(x, edge_index, src, dst, W1l, b1l, W1r, W2l, b2l, W2r):
    f32 = jnp.float32
    W1le = jnp.zeros((_W1, _D), f32).at[:_H].set(W1l)
    e1 = jnp.zeros((1, _W1), f32).at[0, _H].set(1.0)

    s_r = edge_index[0].reshape(_NW, _NCHK, _CH)
    d_r = edge_index[1].reshape(_NW, _NCHK, _CH)
    zeros1 = jnp.zeros((_NP, _W1), f32)
    zeros2 = jnp.zeros((_NP, _H), f32)

    yext, r1 = _tc_dense1(x, W1le, e1, W1r, b1l.reshape(1, _H))
    a10, a11 = _segsum80(s_r, d_r, yext, zeros1)
    y2, r2e = _tc_dense2(a10, a11, r1, W2l, b2l.reshape(1, _H), W2r)
    a20, a21 = _segsum64(s_r, d_r, y2, zeros2)
    return _decode(src.reshape(_NW, 2, 128), dst.reshape(_NW, 2, 128),
                   a20, a21, r2e)


# speedup vs baseline: 1.1202x; 1.0412x over previous
import functools

import jax
import jax.numpy as jnp
from jax import lax
from jax.experimental import pallas as pl
from jax.experimental.pallas import tpu as pltpu
from jax.experimental.pallas import tpu_sc as plsc

_N = 10000
_E = 320000
_D = 128
_H = 64
_B = 8192

_W1 = 80
_NC = 2
_NS = 16
_NW = _NC * _NS
_EPW = _E // _NW
_CH = 80
_NCHK = _EPW // _CH
_NP = 10240
_RPT = _NP // _NS
_BPW = _B // _NW
_NBUF = 5

_mesh = plsc.VectorSubcoreMesh(core_axis_name="c", subcore_axis_name="s")


def _make_segsum(width, with_cnt=False):
    outs = [jax.ShapeDtypeStruct((_NP, width), jnp.float32)] * 2
    if with_cnt:
        outs += [jax.ShapeDtypeStruct((_NP // 16, 16), jnp.float32)] * 2
    scratch = [
        pltpu.VMEM((_NCHK, _CH), jnp.int32),
        pltpu.VMEM((_NCHK, _CH), jnp.int32),
        pltpu.VMEM((_NBUF, _CH, width), jnp.float32),
        pltpu.VMEM_SHARED((_NP, width), jnp.float32),
        pltpu.SemaphoreType.DMA((_NBUF,)),
        pltpu.SemaphoreType.DMA((_NBUF,)),
    ]
    if with_cnt:
        scratch += [
            pltpu.VMEM((_NP // 16, 16), jnp.float32),
            pltpu.VMEM((5, 128), jnp.int32),
            pltpu.VMEM((_NP // 16 // _NS, 16), jnp.float32),
            pltpu.VMEM_SHARED((_NP // 16, 16), jnp.float32),
        ]

    @functools.partial(
        pl.kernel,
        out_type=tuple(outs),
        mesh=_mesh,
        compiler_params=pltpu.CompilerParams(
            use_tc_tiling_on_sc=False, needs_layout_passes=False),
        scratch_types=scratch,
    )
    def seg(s_hbm, d_hbm, tab_hbm, zeros_hbm, *rest):
        if with_cnt:
            (out0_hbm, out1_hbm, outc0_hbm, outc1_hbm,
             sidx_v, didx_v, rows_v, acc_sh, gsem, ssem,
             cnt_v, idb_v, zc_v, accc_sh) = rest
        else:
            (out0_hbm, out1_hbm,
             sidx_v, didx_v, rows_v, acc_sh, gsem, ssem) = rest
        cid = lax.axis_index("c")
        sid = lax.axis_index("s")
        wid = cid * _NS + sid
        pltpu.sync_copy(zeros_hbm.at[pl.ds(sid * _RPT, _RPT)],
                        acc_sh.at[pl.ds(sid * _RPT, _RPT)])
        pltpu.sync_copy(s_hbm.at[wid], sidx_v)
        pltpu.sync_copy(d_hbm.at[wid], didx_v)
        if with_cnt:
            zeros16 = jnp.zeros((16,), jnp.float32)
            lanes16 = lax.iota(jnp.int32, 16)
            nh = _NP // 16
            nz = nh // _NS

            def zh(i, c):
                cnt_v[i, pl.ds(0, 16)] = zeros16
                return c

            lax.fori_loop(0, nh, zh, 0)

            def ib(g, c):
                idb_v[lax.shift_right_logical(g, 3),
                      pl.ds(lax.bitwise_and(g, 7) * 16, 16)] = g * 16 + lanes16
                return c

            lax.fori_loop(0, nh // 16, ib, 0)

            def zz(i, c):
                zc_v[i, pl.ds(0, 16)] = zeros16
                return c

            lax.fori_loop(0, nz, zz, 0)
            pltpu.sync_copy(zc_v, accc_sh.at[pl.ds(sid * nz, nz)])
        plsc.subcore_barrier()

        ones16 = jnp.ones((16,), jnp.float32)

        for b in range(_NBUF):
            pltpu.async_copy(tab_hbm.at[sidx_v.at[b]], rows_v.at[b], gsem.at[b])

        def outer(t, carry):
            j0 = t * _NBUF
            for b in range(_NBUF):
                pltpu.make_async_copy(
                    tab_hbm.at[sidx_v.at[j0 + b]], rows_v.at[b], gsem.at[b]).wait()
                pltpu.async_copy(
                    rows_v.at[b], acc_sh.at[didx_v.at[j0 + b]], ssem.at[b], add=True)
                if with_cnt:
                    for g in range(_CH // 16):
                        dd = didx_v[j0 + b, pl.ds(g * 16, 16)]
                        plsc.addupdate_scatter(
                            cnt_v,
                            [lax.shift_right_logical(dd, 4),
                             lax.bitwise_and(dd, 15)],
                            ones16)
            for b in range(_NBUF):
                nj = j0 + _NBUF + b

                @pl.when(nj < _NCHK)
                def _():
                    pltpu.make_async_copy(
                        rows_v.at[b], acc_sh.at[didx_v.at[j0 + b]], ssem.at[b]).wait()
                    pltpu.async_copy(
                        tab_hbm.at[sidx_v.at[nj]], rows_v.at[b], gsem.at[b])
            return carry

        lax.fori_loop(0, _NCHK // _NBUF, outer, 0)
        jlast = _NCHK - _NBUF
        for b in range(_NBUF):
            pltpu.make_async_copy(
                rows_v.at[b], acc_sh.at[didx_v.at[jlast + b]], ssem.at[b]).wait()
        if with_cnt:
            for k in range(5):
                pltpu.sync_copy(cnt_v.at[pl.ds(k * 128, 128)],
                                accc_sh.at[idb_v.at[k]], add=True)
        plsc.subcore_barrier()

        @pl.when(cid == 0)
        def _():
            pltpu.sync_copy(acc_sh.at[pl.ds(sid * _RPT, _RPT)],
                            out0_hbm.at[pl.ds(sid * _RPT, _RPT)])
            if with_cnt:
                nz = _NP // 16 // _NS
                pltpu.sync_copy(accc_sh.at[pl.ds(sid * nz, nz)],
                                outc0_hbm.at[pl.ds(sid * nz, nz)])

        @pl.when(cid == 1)
        def _():
            pltpu.sync_copy(acc_sh.at[pl.ds(sid * _RPT, _RPT)],
                            out1_hbm.at[pl.ds(sid * _RPT, _RPT)])
            if with_cnt:
                nz = _NP // 16 // _NS
                pltpu.sync_copy(accc_sh.at[pl.ds(sid * nz, nz)],
                                outc1_hbm.at[pl.ds(sid * nz, nz)])

    return seg


_segsum64c = _make_segsum(_H, with_cnt=True)
_segsum64 = _make_segsum(_H)

@functools.partial(
    pl.kernel,
    out_type=jax.ShapeDtypeStruct((_B,), jnp.float32),
    mesh=_mesh,
    compiler_params=pltpu.CompilerParams(
        use_tc_tiling_on_sc=False, needs_layout_passes=False),
    scratch_types=[
        pltpu.VMEM((2, 128), jnp.int32),
        pltpu.VMEM((2, 128), jnp.int32),
        pltpu.VMEM((_BPW, _H), jnp.float32),
        pltpu.VMEM((_BPW, _H), jnp.float32),
        pltpu.VMEM((_BPW, _W1), jnp.float32),
        pltpu.VMEM((_BPW, _H), jnp.float32),
        pltpu.VMEM((_BPW, _H), jnp.float32),
        pltpu.VMEM((_BPW, _W1), jnp.float32),
        pltpu.VMEM((_BPW, 16), jnp.float32),
        pltpu.VMEM((_BPW,), jnp.float32),
        pltpu.SemaphoreType.DMA((12,)),
    ],
)
def _decode(si_hbm, di_hbm, p0_hbm, p1_hbm, r2e_hbm, out_hbm,
            si_v, di_v, as0_v, as1_v, rs_v, ad0_v, ad1_v, rd_v, stage_v, o_v,
            sems):
    cid = lax.axis_index("c")
    sid = lax.axis_index("s")
    wid = cid * _NS + sid
    pltpu.sync_copy(si_hbm.at[wid], si_v)
    pltpu.sync_copy(di_hbm.at[wid], di_v)

    def _copies():
        cps = []
        for t in range(2):
            sl = pl.ds(t * 128, 128)
            cps += [(p0_hbm.at[si_v.at[t]], as0_v.at[sl]),
                    (p1_hbm.at[si_v.at[t]], as1_v.at[sl]),
                    (r2e_hbm.at[si_v.at[t]], rs_v.at[sl]),
                    (p0_hbm.at[di_v.at[t]], ad0_v.at[sl]),
                    (p1_hbm.at[di_v.at[t]], ad1_v.at[sl]),
                    (r2e_hbm.at[di_v.at[t]], rd_v.at[sl])]
        return cps

    for i, (s, d) in enumerate(_copies()):
        pltpu.async_copy(s, d, sems.at[i])
    for i, (s, d) in enumerate(_copies()):
        pltpu.make_async_copy(s, d, sems.at[i]).wait()

    def zrow(p, carry):
        ivs = rs_v[p, pl.ds(_H, 16)]
        ivd = rd_v[p, pl.ds(_H, 16)]
        vs = jnp.zeros((16,), jnp.float32) + ivs[0]
        vd = jnp.zeros((16,), jnp.float32) + ivd[0]
        for c in range(_H // 16):
            sl = pl.ds(c * 16, 16)
            zs = (as0_v[p, sl] + as1_v[p, sl]) * vs + rs_v[p, sl]
            as0_v[p, sl] = jnp.maximum(zs, 0.0)
            zd = (ad0_v[p, sl] + ad1_v[p, sl]) * vd + rd_v[p, sl]
            ad0_v[p, sl] = jnp.maximum(zd, 0.0)
        return carry

    lax.fori_loop(0, _BPW, zrow, 0)

    def body(p, carry):
        t = ((as0_v[p, pl.ds(0, 16)] * ad0_v[p, pl.ds(0, 16)]
              + as0_v[p, pl.ds(16, 16)] * ad0_v[p, pl.ds(16, 16)])
             + (as0_v[p, pl.ds(32, 16)] * ad0_v[p, pl.ds(32, 16)]
                + as0_v[p, pl.ds(48, 16)] * ad0_v[p, pl.ds(48, 16)]))
        stage_v[p, pl.ds(0, 16)] = jnp.cumsum(t)
        return carry

    lax.fori_loop(0, _BPW, body, 0)

    lanes = lax.iota(jnp.int32, 16)
    c15 = jnp.zeros((16,), jnp.int32) + 15

    def sig(g, carry):
        v = plsc.load_gather(stage_v, [g * 16 + lanes, c15])
        o_v[pl.ds(g * 16, 16)] = 1.0 / (1.0 + jnp.exp(-v))
        return carry

    lax.fori_loop(0, _BPW // 16, sig, 0)
    pltpu.sync_copy(o_v, out_hbm.at[pl.ds(wid * _BPW, _BPW)])


_CT = (((1,), (1,)), ((), ()))


def _tc_dense1(x, W1l, W1r, b1):
    g = 10
    bn = _N // g

    def body(x_ref, wl_ref, wr_ref, b1_ref, y1_ref, r1_ref):
        xb = x_ref[...]
        y1_ref[...] = lax.dot_general(
            xb, wl_ref[...], _CT, preferred_element_type=jnp.float32)
        r1_ref[...] = lax.dot_general(
            xb, wr_ref[...], _CT, preferred_element_type=jnp.float32) + b1_ref[...]

    return pl.pallas_call(
        body,
        grid=(g,),
        in_specs=[pl.BlockSpec((bn, _D), lambda i: (i, 0)),
                  pl.BlockSpec((_H, _D), lambda i: (0, 0)),
                  pl.BlockSpec((_H, _D), lambda i: (0, 0)),
                  pl.BlockSpec((1, _H), lambda i: (0, 0))],
        out_specs=[pl.BlockSpec((bn, _H), lambda i: (i, 0)),
                   pl.BlockSpec((bn, _H), lambda i: (i, 0))],
        out_shape=[jax.ShapeDtypeStruct((_N, _H), jnp.float32),
                   jax.ShapeDtypeStruct((_N, _H), jnp.float32)],
    )(x, W1l, W1r, b1)


def _tc_dense2(a0, a1, cnt, r1, W2l, b2, W2r):
    g = 10
    bn = _N // g

    def body(a0_ref, a1_ref, cnt_ref, r1_ref, wl_ref, b_ref, wr_ref,
             y2_ref, r2e_ref):
        agg = a0_ref[...] + a1_ref[...]
        inv = 1.0 / jnp.maximum(cnt_ref[...], 1.0)
        h = jnp.maximum(agg * inv + r1_ref[...], 0.0)
        y2_ref[...] = lax.dot_general(
            h, wl_ref[...], _CT, preferred_element_type=jnp.float32)
        r2 = lax.dot_general(
            h, wr_ref[...], _CT, preferred_element_type=jnp.float32) + b_ref[...]
        lane = lax.broadcasted_iota(jnp.int32, (bn, _W1 - _H), 1)
        ext = jnp.where(lane == 0, inv, 0.0)
        r2e_ref[...] = jnp.concatenate([r2, ext], axis=1)

    return pl.pallas_call(
        body,
        grid=(g,),
        in_specs=[pl.BlockSpec((bn, _H), lambda i: (i, 0)),
                  pl.BlockSpec((bn, _H), lambda i: (i, 0)),
                  pl.BlockSpec((bn, 1), lambda i: (i, 0)),
                  pl.BlockSpec((bn, _H), lambda i: (i, 0)),
                  pl.BlockSpec((_H, _H), lambda i: (0, 0)),
                  pl.BlockSpec((1, _H), lambda i: (0, 0)),
                  pl.BlockSpec((_H, _H), lambda i: (0, 0))],
        out_specs=[pl.BlockSpec((bn, _H), lambda i: (i, 0)),
                   pl.BlockSpec((bn, _W1), lambda i: (i, 0))],
        out_shape=[jax.ShapeDtypeStruct((_N, _H), jnp.float32),
                   jax.ShapeDtypeStruct((_NP, _W1), jnp.float32)],
    )(a0, a1, cnt, r1, W2l, b2, W2r)


def kernel(x, edge_index, src, dst, W1l, b1l, W1r, W2l, b2l, W2r):
    f32 = jnp.float32
    s_r = edge_index[0].reshape(_NW, _NCHK, _CH)
    d_r = edge_index[1].reshape(_NW, _NCHK, _CH)
    zeros = jnp.zeros((_NP, _H), f32)

    y1, r1 = _tc_dense1(x, W1l, W1r, b1l.reshape(1, _H))
    a10, a11, c0, c1 = _segsum64c(s_r, d_r, y1, zeros)
    cnt = (c0 + c1).reshape(_NP, 1)[:_N]
    y2, r2e = _tc_dense2(a10, a11, cnt, r1, W2l, b2l.reshape(1, _H), W2r)
    a20, a21 = _segsum64(s_r, d_r, y2, zeros)
    return _decode(src.reshape(_NW, 2, 128), dst.reshape(_NW, 2, 128),
                   a20, a21, r2e)
